# Initial kernel scaffold; baseline (speedup 1.0000x reference)
import jax
import jax.numpy as jnp
from jax.experimental import pallas as pl


def kernel(x, edge_index, Wl0, Wr0, b0, Wl1, Wr1, b1):
    raise NotImplementedError("write your pallas kernel here")


# speedup vs baseline: 17.1518x; 17.1518x over previous
import functools

import jax
import jax.numpy as jnp
from jax import lax
from jax.experimental import pallas as pl
from jax.experimental.pallas import tpu as pltpu
from jax.experimental.pallas import tpu_sc as plsc

N = 10000
E = 160000
D_IN = 1433
H = 32

NC, NS = 2, 16
IDXW = 128
EPAD = 163840
ROWS_PER_W = EPAD // (NC * NS * IDXW)
NPAD = 10240
NPW = NPAD // NS
DUMMY = NPAD - 1

BM1 = 1000
BM2 = 2000


def _make_sc_scatter(D: int):
  mesh = plsc.VectorSubcoreMesh(
      core_axis_name="c", subcore_axis_name="s", num_cores=NC, num_subcores=NS)

  @functools.partial(
      pl.kernel,
      out_type=jax.ShapeDtypeStruct((NC * NPAD, D), jnp.float32),
      mesh=mesh,
      compiler_params=pltpu.CompilerParams(use_tc_tiling_on_sc=False),
      scratch_types=[
          pltpu.VMEM((ROWS_PER_W, IDXW), jnp.int32),
          pltpu.VMEM((ROWS_PER_W, IDXW), jnp.int32),
          pltpu.VMEM((IDXW, D), jnp.float32),
          pltpu.VMEM_SHARED((NPAD, D), jnp.float32),
          pltpu.SemaphoreType.DMA,
      ],
  )
  def sc_kernel(table_hbm, src_hbm, dst_hbm, zeros_hbm, out_hbm,
                src_v, dst_v, rows_v, acc_sh, sem):
    cid = lax.axis_index("c")
    sid = lax.axis_index("s")
    gid = cid * NS + sid
    pltpu.sync_copy(zeros_hbm.at[pl.ds(sid * NPW, NPW)],
                    acc_sh.at[pl.ds(sid * NPW, NPW)])
    pltpu.sync_copy(src_hbm.at[pl.ds(gid * ROWS_PER_W, ROWS_PER_W)], src_v)
    pltpu.sync_copy(dst_hbm.at[pl.ds(gid * ROWS_PER_W, ROWS_PER_W)], dst_v)
    plsc.subcore_barrier()

    def body(j, carry):
      pltpu.async_copy(table_hbm.at[src_v.at[j]], rows_v, sem).wait()
      pltpu.sync_copy(rows_v, acc_sh.at[dst_v.at[j]], add=True)
      return carry

    lax.fori_loop(0, ROWS_PER_W, body, 0)
    plsc.subcore_barrier()
    pltpu.sync_copy(acc_sh.at[pl.ds(sid * NPW, NPW)],
                    out_hbm.at[pl.ds(cid * NPAD + sid * NPW, NPW)])

  return sc_kernel


_sc_scatter48 = _make_sc_scatter(48)
_sc_scatter32 = _make_sc_scatter(32)


def _tc1_body(x_ref, w_ref, tb_ref, b0_ref, t_out, r_out):
  z = jnp.dot(x_ref[...], w_ref[...], preferred_element_type=jnp.float32)
  t_out[...] = z[:, :48] + tb_ref[...]
  r_out[...] = z[:, 48:] + b0_ref[...]


def _tc1(x, wcat, tb, b0):
  grid = N // BM1
  return pl.pallas_call(
      _tc1_body,
      grid=(grid,),
      in_specs=[
          pl.BlockSpec((BM1, D_IN), lambda i: (i, 0)),
          pl.BlockSpec((D_IN, 80), lambda i: (0, 0)),
          pl.BlockSpec((1, 48), lambda i: (0, 0)),
          pl.BlockSpec((1, 32), lambda i: (0, 0)),
      ],
      out_specs=[
          pl.BlockSpec((BM1, 48), lambda i: (i, 0)),
          pl.BlockSpec((BM1, 32), lambda i: (i, 0)),
      ],
      out_shape=[
          jax.ShapeDtypeStruct((N, 48), jnp.float32),
          jax.ShapeDtypeStruct((N, 32), jnp.float32),
      ],
  )(x, wcat, tb, b0)


def _tc2_body(pa_ref, pb_ref, r0_ref, wl1_ref, wr1_ref, b1_ref,
              t_out, r_out, inv_out):
  s = pa_ref[...] + pb_ref[...]
  cnt = s[:, 32:33]
  inv = 1.0 / jnp.maximum(cnt, 1.0)
  h = s[:, :32] * inv + r0_ref[...]
  t_out[...] = jnp.dot(h, wl1_ref[...], preferred_element_type=jnp.float32)
  r_out[...] = jnp.dot(h, wr1_ref[...], preferred_element_type=jnp.float32) \
      + b1_ref[...]
  inv_out[...] = jnp.broadcast_to(inv, (BM2, 32))


def _tc2(pa, pb, r0, wl1, wr1, b1):
  grid = N // BM2
  return pl.pallas_call(
      _tc2_body,
      grid=(grid,),
      in_specs=[
          pl.BlockSpec((BM2, 48), lambda i: (i, 0)),
          pl.BlockSpec((BM2, 48), lambda i: (i, 0)),
          pl.BlockSpec((BM2, 32), lambda i: (i, 0)),
          pl.BlockSpec((32, 32), lambda i: (0, 0)),
          pl.BlockSpec((32, 32), lambda i: (0, 0)),
          pl.BlockSpec((1, 32), lambda i: (0, 0)),
      ],
      out_specs=[
          pl.BlockSpec((BM2, 32), lambda i: (i, 0)),
          pl.BlockSpec((BM2, 32), lambda i: (i, 0)),
          pl.BlockSpec((BM2, 32), lambda i: (i, 0)),
      ],
      out_shape=[
          jax.ShapeDtypeStruct((N, 32), jnp.float32),
          jax.ShapeDtypeStruct((N, 32), jnp.float32),
          jax.ShapeDtypeStruct((N, 32), jnp.float32),
      ],
  )(pa, pb, r0, wl1, wr1, b1)


def _tc3_body(pa_ref, pb_ref, r1_ref, inv_ref, out_ref):
  h = (pa_ref[...] + pb_ref[...]) * inv_ref[...] + r1_ref[...]
  z = jnp.maximum(h, 0.0)
  m = jnp.max(z, axis=1, keepdims=True)
  lse = jnp.log(jnp.sum(jnp.exp(z - m), axis=1, keepdims=True)) + m
  out_ref[...] = z - lse


def _tc3(pa, pb, r1, inv):
  grid = N // BM2
  return pl.pallas_call(
      _tc3_body,
      grid=(grid,),
      in_specs=[
          pl.BlockSpec((BM2, 32), lambda i: (i, 0)),
          pl.BlockSpec((BM2, 32), lambda i: (i, 0)),
          pl.BlockSpec((BM2, 32), lambda i: (i, 0)),
          pl.BlockSpec((BM2, 32), lambda i: (i, 0)),
      ],
      out_specs=pl.BlockSpec((BM2, 32), lambda i: (i, 0)),
      out_shape=jax.ShapeDtypeStruct((N, 32), jnp.float32),
  )(pa, pb, r1, inv)


def kernel(x, edge_index, Wl0, Wr0, b0, Wl1, Wr1, b1):
  src = edge_index[0].astype(jnp.int32)
  dst = edge_index[1].astype(jnp.int32)
  src_rs = jnp.pad(src, (0, EPAD - E)).reshape(EPAD // IDXW, IDXW)
  dst_rs = jnp.pad(dst, (0, EPAD - E),
                   constant_values=DUMMY).reshape(EPAD // IDXW, IDXW)
  wcat = jnp.concatenate(
      [Wl0, jnp.zeros((D_IN, 16), jnp.float32), Wr0], axis=1)
  tb = jnp.zeros((1, 48), jnp.float32).at[0, 32].set(1.0)
  zeros48 = jnp.zeros((NPAD, 48), jnp.float32)
  zeros32 = jnp.zeros((NPAD, 32), jnp.float32)

  table0, r0 = _tc1(x, wcat, tb, b0.reshape(1, 32))

  p0 = _sc_scatter48(table0, src_rs, dst_rs, zeros48)
  table1, r1, inv = _tc2(p0[:N], p0[NPAD:NPAD + N], r0, Wl1, Wr1,
                         b1.reshape(1, 32))

  p1 = _sc_scatter32(table1, src_rs, dst_rs, zeros32)
  return _tc3(p1[:N], p1[NPAD:NPAD + N], r1, inv)


# speedup vs baseline: 18.1782x; 1.0598x over previous
import functools

import jax
import jax.numpy as jnp
from jax import lax
from jax.experimental import pallas as pl
from jax.experimental.pallas import tpu as pltpu
from jax.experimental.pallas import tpu_sc as plsc

N = 10000
E = 160000
D_IN = 1433
H = 32

NC, NS = 2, 16
IDXW = 128
EPAD = 163840
ROWS_PER_W = EPAD // (NC * NS * IDXW)
NPAD = 10240
NPW = NPAD // NS
DUMMY = NPAD - 1

BM1 = 1000
BM2 = 2000


def _make_sc_scatter(D: int):
  mesh = plsc.VectorSubcoreMesh(
      core_axis_name="c", subcore_axis_name="s", num_cores=NC, num_subcores=NS)

  @functools.partial(
      pl.kernel,
      out_type=jax.ShapeDtypeStruct((NC * NPAD, D), jnp.float32),
      mesh=mesh,
      compiler_params=pltpu.CompilerParams(use_tc_tiling_on_sc=False),
      scratch_types=[
          pltpu.VMEM((ROWS_PER_W, IDXW), jnp.int32),
          pltpu.VMEM((ROWS_PER_W, IDXW), jnp.int32),
          pltpu.VMEM((2, IDXW, D), jnp.float32),
          pltpu.VMEM_SHARED((NPAD, D), jnp.float32),
          pltpu.SemaphoreType.DMA,
      ],
  )
  def sc_kernel(table_hbm, src_hbm, dst_hbm, zeros_hbm, out_hbm,
                src_v, dst_v, rows_v, acc_sh, sem):
    cid = lax.axis_index("c")
    sid = lax.axis_index("s")
    gid = cid * NS + sid
    pltpu.sync_copy(zeros_hbm.at[pl.ds(sid * NPW, NPW)],
                    acc_sh.at[pl.ds(sid * NPW, NPW)])
    pltpu.sync_copy(src_hbm.at[pl.ds(gid * ROWS_PER_W, ROWS_PER_W)], src_v)
    pltpu.sync_copy(dst_hbm.at[pl.ds(gid * ROWS_PER_W, ROWS_PER_W)], dst_v)
    plsc.subcore_barrier()

    def _gather(j, b):
      return pltpu.make_async_copy(
          table_hbm.at[src_v.at[j]], rows_v.at[b], sem)

    _gather(0, 0).start()

    def body(j, carry):
      b = lax.rem(j, 2)
      _gather(j, b).wait()

      @pl.when(j + 1 < ROWS_PER_W)
      def _():
        _gather(j + 1, 1 - b).start()

      pltpu.sync_copy(rows_v.at[b], acc_sh.at[dst_v.at[j]], add=True)
      return carry

    lax.fori_loop(0, ROWS_PER_W, body, 0)
    plsc.subcore_barrier()
    pltpu.sync_copy(acc_sh.at[pl.ds(sid * NPW, NPW)],
                    out_hbm.at[pl.ds(cid * NPAD + sid * NPW, NPW)])

  return sc_kernel


_sc_scatter48 = _make_sc_scatter(48)
_sc_scatter32 = _make_sc_scatter(32)


def _tc1_body(x_ref, w_ref, tb_ref, b0_ref, t_out, r_out):
  z = jnp.dot(x_ref[...], w_ref[...], preferred_element_type=jnp.float32)
  t_out[...] = z[:, :48] + tb_ref[...]
  r_out[...] = z[:, 48:] + b0_ref[...]


def _tc1(x, wcat, tb, b0):
  grid = N // BM1
  return pl.pallas_call(
      _tc1_body,
      grid=(grid,),
      in_specs=[
          pl.BlockSpec((BM1, D_IN), lambda i: (i, 0)),
          pl.BlockSpec((D_IN, 80), lambda i: (0, 0)),
          pl.BlockSpec((1, 48), lambda i: (0, 0)),
          pl.BlockSpec((1, 32), lambda i: (0, 0)),
      ],
      out_specs=[
          pl.BlockSpec((BM1, 48), lambda i: (i, 0)),
          pl.BlockSpec((BM1, 32), lambda i: (i, 0)),
      ],
      out_shape=[
          jax.ShapeDtypeStruct((N, 48), jnp.float32),
          jax.ShapeDtypeStruct((N, 32), jnp.float32),
      ],
  )(x, wcat, tb, b0)


def _tc2_body(pa_ref, pb_ref, r0_ref, wl1_ref, wr1_ref, b1_ref,
              t_out, r_out, inv_out):
  s = pa_ref[...] + pb_ref[...]
  cnt = s[:, 32:33]
  inv = 1.0 / jnp.maximum(cnt, 1.0)
  h = s[:, :32] * inv + r0_ref[...]
  t_out[...] = jnp.dot(h, wl1_ref[...], preferred_element_type=jnp.float32)
  r_out[...] = jnp.dot(h, wr1_ref[...], preferred_element_type=jnp.float32) \
      + b1_ref[...]
  inv_out[...] = jnp.broadcast_to(inv, (BM2, 32))


def _tc2(pa, pb, r0, wl1, wr1, b1):
  grid = N // BM2
  return pl.pallas_call(
      _tc2_body,
      grid=(grid,),
      in_specs=[
          pl.BlockSpec((BM2, 48), lambda i: (i, 0)),
          pl.BlockSpec((BM2, 48), lambda i: (i, 0)),
          pl.BlockSpec((BM2, 32), lambda i: (i, 0)),
          pl.BlockSpec((32, 32), lambda i: (0, 0)),
          pl.BlockSpec((32, 32), lambda i: (0, 0)),
          pl.BlockSpec((1, 32), lambda i: (0, 0)),
      ],
      out_specs=[
          pl.BlockSpec((BM2, 32), lambda i: (i, 0)),
          pl.BlockSpec((BM2, 32), lambda i: (i, 0)),
          pl.BlockSpec((BM2, 32), lambda i: (i, 0)),
      ],
      out_shape=[
          jax.ShapeDtypeStruct((N, 32), jnp.float32),
          jax.ShapeDtypeStruct((N, 32), jnp.float32),
          jax.ShapeDtypeStruct((N, 32), jnp.float32),
      ],
  )(pa, pb, r0, wl1, wr1, b1)


def _tc3_body(pa_ref, pb_ref, r1_ref, inv_ref, out_ref):
  h = (pa_ref[...] + pb_ref[...]) * inv_ref[...] + r1_ref[...]
  z = jnp.maximum(h, 0.0)
  m = jnp.max(z, axis=1, keepdims=True)
  lse = jnp.log(jnp.sum(jnp.exp(z - m), axis=1, keepdims=True)) + m
  out_ref[...] = z - lse


def _tc3(pa, pb, r1, inv):
  grid = N // BM2
  return pl.pallas_call(
      _tc3_body,
      grid=(grid,),
      in_specs=[
          pl.BlockSpec((BM2, 32), lambda i: (i, 0)),
          pl.BlockSpec((BM2, 32), lambda i: (i, 0)),
          pl.BlockSpec((BM2, 32), lambda i: (i, 0)),
          pl.BlockSpec((BM2, 32), lambda i: (i, 0)),
      ],
      out_specs=pl.BlockSpec((BM2, 32), lambda i: (i, 0)),
      out_shape=jax.ShapeDtypeStruct((N, 32), jnp.float32),
  )(pa, pb, r1, inv)


def kernel(x, edge_index, Wl0, Wr0, b0, Wl1, Wr1, b1):
  src = edge_index[0].astype(jnp.int32)
  dst = edge_index[1].astype(jnp.int32)
  src_rs = jnp.pad(src, (0, EPAD - E)).reshape(EPAD // IDXW, IDXW)
  dst_rs = jnp.pad(dst, (0, EPAD - E),
                   constant_values=DUMMY).reshape(EPAD // IDXW, IDXW)
  wcat = jnp.concatenate(
      [Wl0, jnp.zeros((D_IN, 16), jnp.float32), Wr0], axis=1)
  tb = jnp.zeros((1, 48), jnp.float32).at[0, 32].set(1.0)
  zeros48 = jnp.zeros((NPAD, 48), jnp.float32)
  zeros32 = jnp.zeros((NPAD, 32), jnp.float32)

  table0, r0 = _tc1(x, wcat, tb, b0.reshape(1, 32))

  p0 = _sc_scatter48(table0, src_rs, dst_rs, zeros48)
  table1, r1, inv = _tc2(p0[:N], p0[NPAD:NPAD + N], r0, Wl1, Wr1,
                         b1.reshape(1, 32))

  p1 = _sc_scatter32(table1, src_rs, dst_rs, zeros32)
  return _tc3(p1[:N], p1[NPAD:NPAD + N], r1, inv)


# speedup vs baseline: 25.3707x; 1.3957x over previous
import functools

import jax
import jax.numpy as jnp
from jax import lax
from jax.experimental import pallas as pl
from jax.experimental.pallas import tpu as pltpu
from jax.experimental.pallas import tpu_sc as plsc

N = 10000
E = 160000
D_IN = 1433
H = 32

NC, NS = 2, 16
IDXW = 128
EPAD = 163840
RT = 40
NPAD = 10240
NPW = NPAD // NS
NSTG = N // NS

BM1 = 1000
BM2 = 2000


def _make_sc_scatter(with_counts: bool):
  mesh = plsc.VectorSubcoreMesh(
      core_axis_name="c", subcore_axis_name="s", num_cores=NC, num_subcores=NS)

  out_type = [jax.ShapeDtypeStruct((NC * NPAD, 32), jnp.float32)]
  scratch_types = [
      pltpu.VMEM((RT, IDXW), jnp.int32),
      pltpu.VMEM((RT, IDXW), jnp.int32),
      pltpu.VMEM((2, IDXW, 32), jnp.float32),
      pltpu.VMEM_SHARED((N, 32), jnp.float32),
      pltpu.VMEM_SHARED((NPAD, 32), jnp.float32),
      pltpu.SemaphoreType.DMA,
  ]
  if with_counts:
    out_type.append(jax.ShapeDtypeStruct((NC * NPAD, 16), jnp.float32))
    scratch_types += [
        pltpu.VMEM((IDXW, 16), jnp.float32),
        pltpu.VMEM_SHARED((NPAD, 16), jnp.float32),
    ]

  def body(refs):
    if with_counts:
      (table_hbm, src_hbm, dst_hbm, zeros32_hbm, zeros16_hbm, ones_hbm,
       sum_out, cnt_out, src_v, dst_v, rows_v, tab_sh, acc_sh, sem,
       ones_v, cnt_sh) = refs
    else:
      (table_hbm, src_hbm, dst_hbm, zeros32_hbm,
       sum_out, src_v, dst_v, rows_v, tab_sh, acc_sh, sem) = refs

    cid = lax.axis_index("c")
    sid = lax.axis_index("s")
    with jax.named_scope("stage"):
      pltpu.sync_copy(table_hbm.at[pl.ds(sid * NSTG, NSTG)],
                      tab_sh.at[pl.ds(sid * NSTG, NSTG)])
      pltpu.sync_copy(zeros32_hbm.at[pl.ds(sid * NPW, NPW)],
                      acc_sh.at[pl.ds(sid * NPW, NPW)])
      if with_counts:
        pltpu.sync_copy(zeros16_hbm.at[pl.ds(sid * NPW, NPW)],
                        cnt_sh.at[pl.ds(sid * NPW, NPW)])
        pltpu.sync_copy(ones_hbm, ones_v)
      b = (cid * NS + sid) * RT
      pltpu.sync_copy(src_hbm.at[pl.ds(b, RT)], src_v)
      pltpu.sync_copy(dst_hbm.at[pl.ds(b, RT)], dst_v)

    with jax.named_scope("bar1"):
      plsc.subcore_barrier()

    def _gather(j, bb):
      return pltpu.make_async_copy(tab_sh.at[src_v.at[j]], rows_v.at[bb], sem)

    _gather(0, 0).start()

    def loop_body(j, carry):
      bb = lax.rem(j, 2)
      _gather(j, bb).wait()

      @pl.when(j + 1 < RT)
      def _():
        _gather(j + 1, 1 - bb).start()

      pltpu.sync_copy(rows_v.at[bb], acc_sh.at[dst_v.at[j]], add=True)
      if with_counts:
        pltpu.sync_copy(ones_v, cnt_sh.at[dst_v.at[j]], add=True)
      return carry

    with jax.named_scope("edges"):
      lax.fori_loop(0, RT, loop_body, 0)
    with jax.named_scope("bar2"):
      plsc.subcore_barrier()
    with jax.named_scope("wb"):
      pltpu.sync_copy(acc_sh.at[pl.ds(sid * NPW, NPW)],
                      sum_out.at[pl.ds(cid * NPAD + sid * NPW, NPW)])
      if with_counts:
        pltpu.sync_copy(cnt_sh.at[pl.ds(sid * NPW, NPW)],
                        cnt_out.at[pl.ds(cid * NPAD + sid * NPW, NPW)])

  if with_counts:
    @functools.partial(
        pl.kernel, out_type=out_type, mesh=mesh,
        compiler_params=pltpu.CompilerParams(use_tc_tiling_on_sc=False),
        scratch_types=scratch_types)
    def sc_kernel(table_hbm, src_hbm, dst_hbm, zeros32_hbm, zeros16_hbm,
                  ones_hbm, sum_out, cnt_out, src_v, dst_v, rows_v, tab_sh,
                  acc_sh, sem, ones_v, cnt_sh):
      body((table_hbm, src_hbm, dst_hbm, zeros32_hbm, zeros16_hbm, ones_hbm,
            sum_out, cnt_out, src_v, dst_v, rows_v, tab_sh, acc_sh, sem,
            ones_v, cnt_sh))
  else:
    @functools.partial(
        pl.kernel, out_type=out_type[0], mesh=mesh,
        compiler_params=pltpu.CompilerParams(use_tc_tiling_on_sc=False),
        scratch_types=scratch_types)
    def sc_kernel(table_hbm, src_hbm, dst_hbm, zeros32_hbm, sum_out,
                  src_v, dst_v, rows_v, tab_sh, acc_sh, sem):
      body((table_hbm, src_hbm, dst_hbm, zeros32_hbm, sum_out,
            src_v, dst_v, rows_v, tab_sh, acc_sh, sem))

  return sc_kernel


_sc_scatter_l1 = _make_sc_scatter(True)
_sc_scatter_l2 = _make_sc_scatter(False)


def _tc1_body(x_ref, w_ref, b0_ref, t_out, r_out):
  z = jnp.dot(x_ref[...], w_ref[...], preferred_element_type=jnp.float32)
  t_out[...] = z[:, :32]
  r_out[...] = z[:, 32:] + b0_ref[...]


def _tc1(x, wcat, b0):
  grid = N // BM1
  return pl.pallas_call(
      _tc1_body,
      grid=(grid,),
      in_specs=[
          pl.BlockSpec((BM1, D_IN), lambda i: (i, 0)),
          pl.BlockSpec((D_IN, 64), lambda i: (0, 0)),
          pl.BlockSpec((1, 32), lambda i: (0, 0)),
      ],
      out_specs=[
          pl.BlockSpec((BM1, 32), lambda i: (i, 0)),
          pl.BlockSpec((BM1, 32), lambda i: (i, 0)),
      ],
      out_shape=[
          jax.ShapeDtypeStruct((N, 32), jnp.float32),
          jax.ShapeDtypeStruct((N, 32), jnp.float32),
      ],
  )(x, wcat, b0)


def _tc2_body(pa_ref, pb_ref, ca_ref, cb_ref, r0_ref, wl1_ref, wr1_ref,
              b1_ref, t_out, r_out, inv_out):
  s = pa_ref[...] + pb_ref[...]
  cnt = ca_ref[...][:, 0:1] + cb_ref[...][:, 0:1]
  inv = 1.0 / jnp.maximum(cnt, 1.0)
  h = s * inv + r0_ref[...]
  t_out[...] = jnp.dot(h, wl1_ref[...], preferred_element_type=jnp.float32)
  r_out[...] = jnp.dot(h, wr1_ref[...], preferred_element_type=jnp.float32) \
      + b1_ref[...]
  inv_out[...] = jnp.broadcast_to(inv, (BM2, 32))


def _tc2(pa, pb, ca, cb, r0, wl1, wr1, b1):
  grid = N // BM2
  return pl.pallas_call(
      _tc2_body,
      grid=(grid,),
      in_specs=[
          pl.BlockSpec((BM2, 32), lambda i: (i, 0)),
          pl.BlockSpec((BM2, 32), lambda i: (i, 0)),
          pl.BlockSpec((BM2, 16), lambda i: (i, 0)),
          pl.BlockSpec((BM2, 16), lambda i: (i, 0)),
          pl.BlockSpec((BM2, 32), lambda i: (i, 0)),
          pl.BlockSpec((32, 32), lambda i: (0, 0)),
          pl.BlockSpec((32, 32), lambda i: (0, 0)),
          pl.BlockSpec((1, 32), lambda i: (0, 0)),
      ],
      out_specs=[
          pl.BlockSpec((BM2, 32), lambda i: (i, 0)),
          pl.BlockSpec((BM2, 32), lambda i: (i, 0)),
          pl.BlockSpec((BM2, 32), lambda i: (i, 0)),
      ],
      out_shape=[
          jax.ShapeDtypeStruct((N, 32), jnp.float32),
          jax.ShapeDtypeStruct((N, 32), jnp.float32),
          jax.ShapeDtypeStruct((N, 32), jnp.float32),
      ],
  )(pa, pb, ca, cb, r0, wl1, wr1, b1)


def _tc3_body(pa_ref, pb_ref, r1_ref, inv_ref, out_ref):
  h = (pa_ref[...] + pb_ref[...]) * inv_ref[...] + r1_ref[...]
  z = jnp.maximum(h, 0.0)
  m = jnp.max(z, axis=1, keepdims=True)
  lse = jnp.log(jnp.sum(jnp.exp(z - m), axis=1, keepdims=True)) + m
  out_ref[...] = z - lse


def _tc3(pa, pb, r1, inv):
  grid = N // BM2
  return pl.pallas_call(
      _tc3_body,
      grid=(grid,),
      in_specs=[
          pl.BlockSpec((BM2, 32), lambda i: (i, 0)),
          pl.BlockSpec((BM2, 32), lambda i: (i, 0)),
          pl.BlockSpec((BM2, 32), lambda i: (i, 0)),
          pl.BlockSpec((BM2, 32), lambda i: (i, 0)),
      ],
      out_specs=pl.BlockSpec((BM2, 32), lambda i: (i, 0)),
      out_shape=jax.ShapeDtypeStruct((N, 32), jnp.float32),
  )(pa, pb, r1, inv)


def kernel(x, edge_index, Wl0, Wr0, b0, Wl1, Wr1, b1):
  src = edge_index[0].astype(jnp.int32)
  dst = edge_index[1].astype(jnp.int32)
  src_rs = jnp.pad(src, (0, EPAD - E)).reshape(EPAD // IDXW, IDXW)
  dummy_dst = N + (jnp.arange(EPAD - E, dtype=jnp.int32) % (NPAD - N))
  dst_rs = jnp.concatenate([dst, dummy_dst]).reshape(EPAD // IDXW, IDXW)
  wcat = jnp.concatenate([Wl0, Wr0], axis=1)
  zeros32 = jnp.zeros((NPAD, 32), jnp.float32)
  zeros16 = jnp.zeros((NPAD, 16), jnp.float32)
  ones16 = jnp.ones((IDXW, 16), jnp.float32)

  table0, r0 = _tc1(x, wcat, b0.reshape(1, 32))

  p0, pc = _sc_scatter_l1(table0, src_rs, dst_rs, zeros32, zeros16, ones16)
  table1, r1, inv = _tc2(p0[:N], p0[NPAD:NPAD + N], pc[:N],
                         pc[NPAD:NPAD + N], r0, Wl1, Wr1, b1.reshape(1, 32))

  p1 = _sc_scatter_l2(table1, src_rs, dst_rs, zeros32)
  return _tc3(p1[:N], p1[NPAD:NPAD + N], r1, inv)


# speedup vs baseline: 36.2190x; 1.4276x over previous
import functools

import jax
import jax.numpy as jnp
from jax import lax
from jax.experimental import pallas as pl
from jax.experimental.pallas import tpu as pltpu
from jax.experimental.pallas import tpu_sc as plsc

N = 10000
E = 160000
D_IN = 1433
H = 32

NC, NS = 2, 16
IDXW = 128
EPAD = 163840
RT = 40
NPAD = 10240
NPW = NPAD // NS
NSTG = N // NS

BM1 = 1024
BM2 = 2048


def _make_sc_scatter(with_counts: bool):
  mesh = plsc.VectorSubcoreMesh(
      core_axis_name="c", subcore_axis_name="s", num_cores=NC, num_subcores=NS)

  out_type = [jax.ShapeDtypeStruct((NC * NPAD, 32), jnp.float32)]
  scratch_types = [
      pltpu.VMEM((RT, IDXW), jnp.int32),
      pltpu.VMEM((RT, IDXW), jnp.int32),
      pltpu.VMEM((2, IDXW, 32), jnp.float32),
      pltpu.VMEM_SHARED((N, 32), jnp.float32),
      pltpu.VMEM_SHARED((NPAD, 32), jnp.float32),
      pltpu.SemaphoreType.DMA,
  ]
  if with_counts:
    out_type.append(jax.ShapeDtypeStruct((NC * NPAD, 16), jnp.float32))
    scratch_types += [
        pltpu.VMEM((IDXW, 16), jnp.float32),
        pltpu.VMEM_SHARED((NPAD, 16), jnp.float32),
    ]

  def body(refs):
    if with_counts:
      (table_hbm, src_hbm, dst_hbm, zeros32_hbm, zeros16_hbm, ones_hbm,
       sum_out, cnt_out, src_v, dst_v, rows_v, tab_sh, acc_sh, sem,
       ones_v, cnt_sh) = refs
    else:
      (table_hbm, src_hbm, dst_hbm, zeros32_hbm,
       sum_out, src_v, dst_v, rows_v, tab_sh, acc_sh, sem) = refs

    cid = lax.axis_index("c")
    sid = lax.axis_index("s")
    with jax.named_scope("stage"):
      pltpu.sync_copy(table_hbm.at[pl.ds(sid * NSTG, NSTG)],
                      tab_sh.at[pl.ds(sid * NSTG, NSTG)])
      pltpu.sync_copy(zeros32_hbm.at[pl.ds(sid * NPW, NPW)],
                      acc_sh.at[pl.ds(sid * NPW, NPW)])
      if with_counts:
        pltpu.sync_copy(zeros16_hbm.at[pl.ds(sid * NPW, NPW)],
                        cnt_sh.at[pl.ds(sid * NPW, NPW)])
        pltpu.sync_copy(ones_hbm, ones_v)
      b = (cid * NS + sid) * RT
      pltpu.sync_copy(src_hbm.at[pl.ds(b, RT)], src_v)
      pltpu.sync_copy(dst_hbm.at[pl.ds(b, RT)], dst_v)

    with jax.named_scope("bar1"):
      plsc.subcore_barrier()

    def _gather(j, bb):
      return pltpu.make_async_copy(tab_sh.at[src_v.at[j]], rows_v.at[bb], sem)

    _gather(0, 0).start()

    def loop_body(j, carry):
      bb = lax.rem(j, 2)
      _gather(j, bb).wait()

      @pl.when(j + 1 < RT)
      def _():
        _gather(j + 1, 1 - bb).start()

      pltpu.sync_copy(rows_v.at[bb], acc_sh.at[dst_v.at[j]], add=True)
      if with_counts:
        pltpu.sync_copy(ones_v, cnt_sh.at[dst_v.at[j]], add=True)
      return carry

    with jax.named_scope("edges"):
      lax.fori_loop(0, RT, loop_body, 0)
    with jax.named_scope("bar2"):
      plsc.subcore_barrier()
    with jax.named_scope("wb"):
      pltpu.sync_copy(acc_sh.at[pl.ds(sid * NPW, NPW)],
                      sum_out.at[pl.ds(cid * NPAD + sid * NPW, NPW)])
      if with_counts:
        pltpu.sync_copy(cnt_sh.at[pl.ds(sid * NPW, NPW)],
                        cnt_out.at[pl.ds(cid * NPAD + sid * NPW, NPW)])

  if with_counts:
    @functools.partial(
        pl.kernel, out_type=out_type, mesh=mesh,
        compiler_params=pltpu.CompilerParams(use_tc_tiling_on_sc=False),
        scratch_types=scratch_types)
    def sc_kernel(table_hbm, src_hbm, dst_hbm, zeros32_hbm, zeros16_hbm,
                  ones_hbm, sum_out, cnt_out, src_v, dst_v, rows_v, tab_sh,
                  acc_sh, sem, ones_v, cnt_sh):
      body((table_hbm, src_hbm, dst_hbm, zeros32_hbm, zeros16_hbm, ones_hbm,
            sum_out, cnt_out, src_v, dst_v, rows_v, tab_sh, acc_sh, sem,
            ones_v, cnt_sh))
  else:
    @functools.partial(
        pl.kernel, out_type=out_type[0], mesh=mesh,
        compiler_params=pltpu.CompilerParams(use_tc_tiling_on_sc=False),
        scratch_types=scratch_types)
    def sc_kernel(table_hbm, src_hbm, dst_hbm, zeros32_hbm, sum_out,
                  src_v, dst_v, rows_v, tab_sh, acc_sh, sem):
      body((table_hbm, src_hbm, dst_hbm, zeros32_hbm, sum_out,
            src_v, dst_v, rows_v, tab_sh, acc_sh, sem))

  return sc_kernel


_sc_scatter_l1 = _make_sc_scatter(True)
_sc_scatter_l2 = _make_sc_scatter(False)


def _tc1_body(xt_ref, w_ref, b0_ref, t_out, r_out):
  z = lax.dot_general(xt_ref[...], w_ref[...], (((0,), (0,)), ((), ())),
                      preferred_element_type=jnp.float32)
  t_out[...] = z[:, :32]
  r_out[...] = z[:, 32:] + b0_ref[...]


def _tc1(xt, wcat, b0):
  grid = NPAD // BM1
  return pl.pallas_call(
      _tc1_body,
      grid=(grid,),
      in_specs=[
          pl.BlockSpec((D_IN, BM1), lambda i: (0, i)),
          pl.BlockSpec((D_IN, 64), lambda i: (0, 0)),
          pl.BlockSpec((1, 32), lambda i: (0, 0)),
      ],
      out_specs=[
          pl.BlockSpec((BM1, 32), lambda i: (i, 0)),
          pl.BlockSpec((BM1, 32), lambda i: (i, 0)),
      ],
      out_shape=[
          jax.ShapeDtypeStruct((NPAD, 32), jnp.float32),
          jax.ShapeDtypeStruct((NPAD, 32), jnp.float32),
      ],
  )(xt, wcat, b0)


def _tc2_body(pa_ref, pb_ref, ca_ref, cb_ref, r0_ref, wl1_ref, wr1_ref,
              b1_ref, t_out, r_out, inv_out):
  s = pa_ref[...] + pb_ref[...]
  cnt = ca_ref[...][:, 0:1] + cb_ref[...][:, 0:1]
  inv = 1.0 / jnp.maximum(cnt, 1.0)
  h = s * inv + r0_ref[...]
  t_out[...] = jnp.dot(h, wl1_ref[...], preferred_element_type=jnp.float32)
  r_out[...] = jnp.dot(h, wr1_ref[...], preferred_element_type=jnp.float32) \
      + b1_ref[...]
  inv_out[...] = jnp.broadcast_to(inv, (BM2, 32))


def _tc2(p0, pc, r0, wl1, wr1, b1):
  grid = NPAD // BM2
  nb = NPAD // BM2
  return pl.pallas_call(
      _tc2_body,
      grid=(grid,),
      in_specs=[
          pl.BlockSpec((BM2, 32), lambda i: (i, 0)),
          pl.BlockSpec((BM2, 32), lambda i: (i + nb, 0)),
          pl.BlockSpec((BM2, 16), lambda i: (i, 0)),
          pl.BlockSpec((BM2, 16), lambda i: (i + nb, 0)),
          pl.BlockSpec((BM2, 32), lambda i: (i, 0)),
          pl.BlockSpec((32, 32), lambda i: (0, 0)),
          pl.BlockSpec((32, 32), lambda i: (0, 0)),
          pl.BlockSpec((1, 32), lambda i: (0, 0)),
      ],
      out_specs=[
          pl.BlockSpec((BM2, 32), lambda i: (i, 0)),
          pl.BlockSpec((BM2, 32), lambda i: (i, 0)),
          pl.BlockSpec((BM2, 32), lambda i: (i, 0)),
      ],
      out_shape=[
          jax.ShapeDtypeStruct((NPAD, 32), jnp.float32),
          jax.ShapeDtypeStruct((NPAD, 32), jnp.float32),
          jax.ShapeDtypeStruct((NPAD, 32), jnp.float32),
      ],
  )(p0, p0, pc, pc, r0, wl1, wr1, b1)


def _tc3_body(pa_ref, pb_ref, r1_ref, inv_ref, out_ref):
  h = (pa_ref[...] + pb_ref[...]) * inv_ref[...] + r1_ref[...]
  z = jnp.maximum(h, 0.0)
  m = jnp.max(z, axis=1, keepdims=True)
  lse = jnp.log(jnp.sum(jnp.exp(z - m), axis=1, keepdims=True)) + m
  out_ref[...] = (z - lse).T


def _tc3(p1, r1, inv):
  grid = NPAD // BM2
  nb = NPAD // BM2
  return pl.pallas_call(
      _tc3_body,
      grid=(grid,),
      in_specs=[
          pl.BlockSpec((BM2, 32), lambda i: (i, 0)),
          pl.BlockSpec((BM2, 32), lambda i: (i + nb, 0)),
          pl.BlockSpec((BM2, 32), lambda i: (i, 0)),
          pl.BlockSpec((BM2, 32), lambda i: (i, 0)),
      ],
      out_specs=pl.BlockSpec((32, BM2), lambda i: (0, i)),
      out_shape=jax.ShapeDtypeStruct((32, N), jnp.float32),
  )(p1, p1, r1, inv)


def kernel(x, edge_index, Wl0, Wr0, b0, Wl1, Wr1, b1):
  src = edge_index[0].astype(jnp.int32)
  dst = edge_index[1].astype(jnp.int32)
  src_rs = jnp.pad(src, (0, EPAD - E)).reshape(EPAD // IDXW, IDXW)
  dummy_dst = N + (jnp.arange(EPAD - E, dtype=jnp.int32) % (NPAD - N))
  dst_rs = jnp.concatenate([dst, dummy_dst]).reshape(EPAD // IDXW, IDXW)
  wcat = jnp.concatenate([Wl0, Wr0], axis=1)
  zeros32 = jnp.zeros((NPAD, 32), jnp.float32)
  zeros16 = jnp.zeros((NPAD, 16), jnp.float32)
  ones16 = jnp.ones((IDXW, 16), jnp.float32)

  table0, r0 = _tc1(x.T, wcat, b0.reshape(1, 32))

  p0, pc = _sc_scatter_l1(table0, src_rs, dst_rs, zeros32, zeros16, ones16)
  table1, r1, inv = _tc2(p0, pc, r0, Wl1, Wr1, b1.reshape(1, 32))

  p1 = _sc_scatter_l2(table1, src_rs, dst_rs, zeros32)
  return _tc3(p1, r1, inv).T
